# baseline (device time: 135746 ns/iter reference)
import jax
import jax.numpy as jnp
from jax import lax
from jax.experimental import pallas as pl
from jax.experimental.pallas import tpu as pltpu

N_DEV = 32
B, SQ, SKV = 2, 512, 512
H_LOC, DH = 8, 64
DM = 768
BLK = 64
ROWS = B * SQ

N_LVL = 5
HALF_ROWS = ROWS // 2
RS_HALF = [256, 128, 64, 32, 16]
AG_SZ = [16, 32, 64, 128, 256]
RS_SOFF = [0, 256, 384, 448, 480]
STAGE_B = 496
AG_BASE = 2 * STAGE_B
ORD = [[0, 1, 2, 3, 4],
       [1, 2, 0, 4, 3]]


def kernel(x, Wq, K_ext, V_ext, Wo):
    x2 = x.reshape(ROWS, DM)
    my = lax.axis_index("i")
    K_loc = lax.dynamic_slice_in_dim(K_ext, (my // 2) * (2 * H_LOC),
                                     2 * H_LOC, axis=2).astype(jnp.bfloat16)
    V_loc = lax.dynamic_slice_in_dim(V_ext, (my // 2) * (2 * H_LOC),
                                     2 * H_LOC, axis=2).astype(jnp.bfloat16)

    def body(x_ref, wq_ref, kbuf, vbuf, wo_ref, out_ref,
             stage_ref, sendbuf, send_sems, recv_sems):
        p = lax.axis_index("i")
        z = p // 8
        q = p % 8
        y = q // 2
        xb = (q % 2) ^ (y % 2)

        def pos_of(xb_, y_, z_):
            return 8 * z_ + 2 * y_ + (xb_ ^ (y_ % 2))

        levels = [
            (xb, pos_of(xb ^ 1, y, z)),
            (y % 2, pos_of(xb, y ^ 1, z)),
            (z % 2, pos_of(xb, y, z ^ 1)),
            (y // 2, pos_of(xb, y ^ 2, z)),
            (z // 2, pos_of(xb, y, z ^ 2)),
        ]

        barrier_sem = pltpu.get_barrier_semaphore()
        for _, partner in levels:
            pl.semaphore_signal(
                barrier_sem, inc=1,
                device_id=(partner,), device_id_type=pl.DeviceIdType.MESH,
            )
        pl.semaphore_wait(barrier_sem, N_LVL)

        rowblk = lax.broadcasted_iota(jnp.int32, (SQ, SKV), 0) // BLK
        colblk = lax.broadcasted_iota(jnp.int32, (SQ, SKV), 1) // BLK
        mask = colblk <= rowblk

        def compute(h_base):
            for b in range(B):
                q_b = jnp.dot(x_ref[pl.ds(b * SQ, SQ), :], wq_ref[...],
                              preferred_element_type=jnp.float32)
                ctxs = []
                for h in range(H_LOC):
                    qh = q_b[:, h * DH:(h + 1) * DH].astype(jnp.bfloat16)
                    kh = kbuf[b, :, h_base + h, :]
                    vh = vbuf[b, :, h_base + h, :]
                    s = lax.dot_general(
                        qh, kh, (((1,), (1,)), ((), ())),
                        preferred_element_type=jnp.float32) * 0.125
                    s = jnp.where(mask, s, -1e9)
                    m = jnp.max(s, axis=1, keepdims=True)
                    pr = jnp.exp(s - m)
                    pr = (pr / jnp.sum(pr, axis=1, keepdims=True)
                          ).astype(jnp.bfloat16)
                    ctxs.append(jnp.dot(pr, vh,
                                        preferred_element_type=jnp.float32))
                ctx = jnp.concatenate(ctxs, axis=1)
                out_ref[pl.ds(b * SQ, SQ), :] = jnp.dot(
                    ctx, wo_ref[...], preferred_element_type=jnp.float32)

        pl.when(p % 2 == 0)(lambda: compute(0))
        pl.when(p % 2 == 1)(lambda: compute(H_LOC))

        offs = [jnp.int32(0), jnp.int32(HALF_ROWS)]
        for s in range(N_LVL):
            half = RS_HALF[s]
            rdmas = []
            for hf in range(2):
                bit, partner = levels[ORD[hf][s]]
                give = pl.multiple_of(offs[hf] + (1 - bit) * half, 8)
                soff = RS_SOFF[s] + hf * STAGE_B
                sendbuf[hf, pl.ds(0, half), :] = (
                    out_ref[pl.ds(give, half), :].astype(jnp.bfloat16))
                rdma = pltpu.make_async_remote_copy(
                    src_ref=sendbuf.at[hf, pl.ds(0, half), :],
                    dst_ref=stage_ref.at[pl.ds(soff, half), :],
                    send_sem=send_sems.at[2 * s + hf],
                    recv_sem=recv_sems.at[2 * s + hf],
                    device_id=(partner,),
                    device_id_type=pl.DeviceIdType.MESH,
                )
                rdma.start()
                rdmas.append(rdma)
                offs[hf] = offs[hf] + bit * half
            for hf in range(2):
                rdmas[hf].wait()
                soff = RS_SOFF[s] + hf * STAGE_B
                keep = pl.multiple_of(offs[hf], 8)
                out_ref[pl.ds(keep, half), :] += (
                    stage_ref[pl.ds(soff, half), :].astype(jnp.float32))

        for s in range(N_LVL):
            sz = AG_SZ[s]
            rdmas = []
            sibs = []
            for hf in range(2):
                bit, partner = levels[ORD[hf][N_LVL - 1 - s]]
                soff = AG_BASE + RS_SOFF[N_LVL - 1 - s] + hf * STAGE_B
                own = pl.multiple_of(offs[hf], 8)
                sendbuf[hf, pl.ds(0, sz), :] = (
                    out_ref[pl.ds(own, sz), :].astype(jnp.bfloat16))
                rdma = pltpu.make_async_remote_copy(
                    src_ref=sendbuf.at[hf, pl.ds(0, sz), :],
                    dst_ref=stage_ref.at[pl.ds(soff, sz), :],
                    send_sem=send_sems.at[10 + 2 * s + hf],
                    recv_sem=recv_sems.at[10 + 2 * s + hf],
                    device_id=(partner,),
                    device_id_type=pl.DeviceIdType.MESH,
                )
                rdma.start()
                rdmas.append(rdma)
                sibs.append(offs[hf] + (1 - 2 * bit) * sz)
                offs[hf] = offs[hf] - bit * sz
            for hf in range(2):
                rdmas[hf].wait()
                soff = AG_BASE + RS_SOFF[N_LVL - 1 - s] + hf * STAGE_B
                sib = pl.multiple_of(sibs[hf], 8)
                out_ref[pl.ds(sib, sz), :] = (
                    stage_ref[pl.ds(soff, sz), :].astype(jnp.float32))

    out2 = pl.pallas_call(
        body,
        out_shape=jax.ShapeDtypeStruct((ROWS, DM), jnp.float32),
        in_specs=[pl.BlockSpec(memory_space=pltpu.VMEM)] * 5,
        out_specs=pl.BlockSpec(memory_space=pltpu.VMEM),
        scratch_shapes=[
            pltpu.VMEM((4 * STAGE_B, DM), jnp.bfloat16),
            pltpu.VMEM((2, 256, DM), jnp.bfloat16),
            pltpu.SemaphoreType.DMA((20,)),
            pltpu.SemaphoreType.DMA((20,)),
        ],
        compiler_params=pltpu.CompilerParams(collective_id=0),
    )(x2, Wq, K_loc, V_loc, Wo)
    return out2.reshape(B, SQ, DM)


# device time: 131915 ns/iter; 1.0290x vs baseline; 1.0290x over previous
import jax
import jax.numpy as jnp
from jax import lax
from jax.experimental import pallas as pl
from jax.experimental.pallas import tpu as pltpu

N_DEV = 32
B, SQ, SKV = 2, 512, 512
H_LOC, DH = 8, 64
DM = 768
BLK = 64
ROWS = B * SQ

N_LVL = 5
HALF_ROWS = ROWS // 2
RS_HALF = [256, 128, 64, 32, 16]
AG_SZ = [16, 32, 64, 128, 256]
RS_SOFF = [0, 256, 384, 448, 480]
STAGE_B = 496
AG_BASE = 2 * STAGE_B
ORD = [[0, 1, 2, 3, 4],
       [1, 2, 0, 4, 3]]


def kernel(x, Wq, K_ext, V_ext, Wo):
    x2 = x.reshape(ROWS, DM)
    my = lax.axis_index("i")
    K_loc = lax.dynamic_slice_in_dim(K_ext, my * H_LOC, H_LOC, axis=2
                                     ).astype(jnp.bfloat16)
    V_loc = lax.dynamic_slice_in_dim(V_ext, my * H_LOC, H_LOC, axis=2
                                     ).astype(jnp.bfloat16)

    def body(x_ref, wq_ref, kbuf, vbuf, wo_ref, out_ref,
             stage_ref, sendbuf, send_sems, recv_sems):
        p = lax.axis_index("i")
        z = p // 8
        q = p % 8
        y = q // 2
        xb = (q % 2) ^ (y % 2)

        def pos_of(xb_, y_, z_):
            return 8 * z_ + 2 * y_ + (xb_ ^ (y_ % 2))

        levels = [
            (xb, pos_of(xb ^ 1, y, z)),
            (y % 2, pos_of(xb, y ^ 1, z)),
            (z % 2, pos_of(xb, y, z ^ 1)),
            (y // 2, pos_of(xb, y ^ 2, z)),
            (z // 2, pos_of(xb, y, z ^ 2)),
        ]

        barrier_sem = pltpu.get_barrier_semaphore()
        for _, partner in levels:
            pl.semaphore_signal(
                barrier_sem, inc=1,
                device_id=(partner,), device_id_type=pl.DeviceIdType.MESH,
            )
        pl.semaphore_wait(barrier_sem, N_LVL)

        rowblk = lax.broadcasted_iota(jnp.int32, (SQ, SKV), 0) // BLK
        colblk = lax.broadcasted_iota(jnp.int32, (SQ, SKV), 1) // BLK
        mask = colblk <= rowblk

        for b in range(B):
            q_b = jnp.dot(x_ref[pl.ds(b * SQ, SQ), :], wq_ref[...],
                          preferred_element_type=jnp.float32)
            ctxs = []
            for h in range(H_LOC):
                qh = q_b[:, h * DH:(h + 1) * DH].astype(jnp.bfloat16)
                kh = kbuf[b, :, h, :]
                vh = vbuf[b, :, h, :]
                s = lax.dot_general(
                    qh, kh, (((1,), (1,)), ((), ())),
                    preferred_element_type=jnp.float32) * 0.125
                s = jnp.where(mask, s, -1e9)
                m = jnp.max(s, axis=1, keepdims=True)
                pr = jnp.exp(s - m)
                pr = (pr / jnp.sum(pr, axis=1, keepdims=True)
                      ).astype(jnp.bfloat16)
                ctxs.append(jnp.dot(pr, vh, preferred_element_type=jnp.float32))
            ctx = jnp.concatenate(ctxs, axis=1)
            out_ref[pl.ds(b * SQ, SQ), :] = jnp.dot(
                ctx, wo_ref[...], preferred_element_type=jnp.float32)

        offs = [jnp.int32(0), jnp.int32(HALF_ROWS)]
        for s in range(N_LVL):
            half = RS_HALF[s]
            rdmas = []
            for hf in range(2):
                bit, partner = levels[ORD[hf][s]]
                give = pl.multiple_of(offs[hf] + (1 - bit) * half, 8)
                soff = RS_SOFF[s] + hf * STAGE_B
                sendbuf[hf, pl.ds(0, half), :] = (
                    out_ref[pl.ds(give, half), :].astype(jnp.bfloat16))
                rdma = pltpu.make_async_remote_copy(
                    src_ref=sendbuf.at[hf, pl.ds(0, half), :],
                    dst_ref=stage_ref.at[pl.ds(soff, half), :],
                    send_sem=send_sems.at[2 * s + hf],
                    recv_sem=recv_sems.at[2 * s + hf],
                    device_id=(partner,),
                    device_id_type=pl.DeviceIdType.MESH,
                )
                rdma.start()
                rdmas.append(rdma)
                offs[hf] = offs[hf] + bit * half
            for hf in range(2):
                rdmas[hf].wait()
                soff = RS_SOFF[s] + hf * STAGE_B
                keep = pl.multiple_of(offs[hf], 8)
                out_ref[pl.ds(keep, half), :] += (
                    stage_ref[pl.ds(soff, half), :].astype(jnp.float32))

        for s in range(N_LVL):
            sz = AG_SZ[s]
            rdmas = []
            sibs = []
            for hf in range(2):
                bit, partner = levels[ORD[hf][N_LVL - 1 - s]]
                soff = AG_BASE + RS_SOFF[N_LVL - 1 - s] + hf * STAGE_B
                own = pl.multiple_of(offs[hf], 8)
                sendbuf[hf, pl.ds(0, sz), :] = (
                    out_ref[pl.ds(own, sz), :].astype(jnp.bfloat16))
                rdma = pltpu.make_async_remote_copy(
                    src_ref=sendbuf.at[hf, pl.ds(0, sz), :],
                    dst_ref=stage_ref.at[pl.ds(soff, sz), :],
                    send_sem=send_sems.at[10 + 2 * s + hf],
                    recv_sem=recv_sems.at[10 + 2 * s + hf],
                    device_id=(partner,),
                    device_id_type=pl.DeviceIdType.MESH,
                )
                rdma.start()
                rdmas.append(rdma)
                sibs.append(offs[hf] + (1 - 2 * bit) * sz)
                offs[hf] = offs[hf] - bit * sz
            for hf in range(2):
                rdmas[hf].wait()
                soff = AG_BASE + RS_SOFF[N_LVL - 1 - s] + hf * STAGE_B
                sib = pl.multiple_of(sibs[hf], 8)
                out_ref[pl.ds(sib, sz), :] = (
                    stage_ref[pl.ds(soff, sz), :].astype(jnp.float32))

    out2 = pl.pallas_call(
        body,
        out_shape=jax.ShapeDtypeStruct((ROWS, DM), jnp.float32),
        in_specs=[pl.BlockSpec(memory_space=pltpu.VMEM)] * 5,
        out_specs=pl.BlockSpec(memory_space=pltpu.VMEM),
        scratch_shapes=[
            pltpu.VMEM((4 * STAGE_B, DM), jnp.bfloat16),
            pltpu.VMEM((2, 256, DM), jnp.bfloat16),
            pltpu.SemaphoreType.DMA((20,)),
            pltpu.SemaphoreType.DMA((20,)),
        ],
        compiler_params=pltpu.CompilerParams(collective_id=0),
    )(x2, Wq, K_loc, V_loc, Wo)
    return out2.reshape(B, SQ, DM)
